# Initial kernel scaffold; baseline (speedup 1.0000x reference)
import jax
import jax.numpy as jnp
from jax.experimental import pallas as pl


def kernel(x, pos, batch, edge_index, edge_weights, lig_flag, chains, params):
    raise NotImplementedError("write your pallas kernel here")


# speedup vs baseline: 1.0228x; 1.0228x over previous
import functools
import jax
import jax.numpy as jnp
from jax import lax
from jax.experimental import pallas as pl
from jax.experimental.pallas import tpu as pltpu
from jax.experimental.pallas import tpu_sc as plsc

N = 50000
E = 800000
SDIM = 64
NUM_RADIAL = 32
DEPTH = 3
CUTOFF = 5.0
G = 8

NSC = 2
NSUB = 16
UNIT = 128
EPT = 51200
E_PAD = NSUB * EPT
UNITS = EPT // UNIT
NHALF = 25000
ROWS_PT = 1568
ACC_ROWS = NSUB * ROWS_PT
NP = 51200
BN = 2048
BE = 4096


def _silu(v):
    return v / (1.0 + jnp.exp(-v))


def _embed_body(x_ref, emb_ref, lin_ref, s_ref, slin_ref):
    xb = x_ref[...]
    iota = lax.broadcasted_iota(jnp.int32, (BN, 128), 1)
    oh = (iota == xb).astype(jnp.float32)
    s = jnp.dot(oh, emb_ref[...], preferred_element_type=jnp.float32)
    s_ref[...] = s
    slin_ref[...] = jnp.dot(s, lin_ref[...], preferred_element_type=jnp.float32)


def _embed_call(x_p, emb_pad, lin0):
    grid = NP // BN
    return pl.pallas_call(
        _embed_body,
        grid=(grid,),
        in_specs=[
            pl.BlockSpec((BN, 1), lambda i: (i, 0)),
            pl.BlockSpec((128, SDIM), lambda i: (0, 0)),
            pl.BlockSpec((SDIM, SDIM), lambda i: (0, 0)),
        ],
        out_specs=[
            pl.BlockSpec((BN, SDIM), lambda i: (i, 0)),
            pl.BlockSpec((BN, SDIM), lambda i: (i, 0)),
        ],
        out_shape=[
            jax.ShapeDtypeStruct((NP, SDIM), jnp.float32),
            jax.ShapeDtypeStruct((NP, SDIM), jnp.float32),
        ],
    )(x_p, emb_pad, lin0)


def _edge_w_body(d_ref, fW1_ref, fb1_ref, fW2_ref, fb2_ref, w_ref):
    d = d_ref[...]
    n = (lax.broadcasted_iota(jnp.int32, (BE, NUM_RADIAL), 1) + 1
         ).astype(jnp.float32)
    arg = n * (jnp.pi / CUTOFF) * d
    rbf = jnp.sqrt(2.0 / CUTOFF) * jnp.sin(arg) / d
    env = 0.5 * (jnp.cos(jnp.pi * d / CUTOFF) + 1.0)
    env = env * (d < CUTOFF).astype(jnp.float32)
    h = _silu(jnp.dot(rbf, fW1_ref[...], preferred_element_type=jnp.float32)
              + fb1_ref[...])
    w = jnp.dot(h, fW2_ref[...], preferred_element_type=jnp.float32) + fb2_ref[...]
    w_ref[...] = w * env


def _edge_w_call(d_p, fW1, fb1, fW2, fb2):
    grid = E_PAD // BE
    return pl.pallas_call(
        _edge_w_body,
        grid=(grid,),
        in_specs=[
            pl.BlockSpec((BE, 1), lambda i: (i, 0)),
            pl.BlockSpec((NUM_RADIAL, SDIM), lambda i: (0, 0)),
            pl.BlockSpec((1, SDIM), lambda i: (0, 0)),
            pl.BlockSpec((SDIM, SDIM), lambda i: (0, 0)),
            pl.BlockSpec((1, SDIM), lambda i: (0, 0)),
        ],
        out_specs=pl.BlockSpec((BE, SDIM), lambda i: (i, 0)),
        out_shape=jax.ShapeDtypeStruct((E_PAD, SDIM), jnp.float32),
    )(d_p, fW1, fb1, fW2, fb2)


def _update_body(s_ref, agg_ref, deg_ref, uW1_ref, ub1_ref, uW2_ref, ub2_ref,
                 lin_ref, s_out_ref, slin_ref):
    deg = jnp.maximum(deg_ref[...], 1.0)
    a = agg_ref[...] / deg
    h = _silu(jnp.dot(a, uW1_ref[...], preferred_element_type=jnp.float32)
              + ub1_ref[...])
    s_new = s_ref[...] + jnp.dot(h, uW2_ref[...],
                                 preferred_element_type=jnp.float32) + ub2_ref[...]
    s_out_ref[...] = s_new
    slin_ref[...] = jnp.dot(s_new, lin_ref[...], preferred_element_type=jnp.float32)


def _update_call(s, agg, deg, uW1, ub1, uW2, ub2, lin_next):
    grid = NP // BN
    return pl.pallas_call(
        _update_body,
        grid=(grid,),
        in_specs=[
            pl.BlockSpec((BN, SDIM), lambda i: (i, 0)),
            pl.BlockSpec((BN, SDIM), lambda i: (i, 0)),
            pl.BlockSpec((BN, 1), lambda i: (i, 0)),
            pl.BlockSpec((SDIM, SDIM), lambda i: (0, 0)),
            pl.BlockSpec((1, SDIM), lambda i: (0, 0)),
            pl.BlockSpec((SDIM, SDIM), lambda i: (0, 0)),
            pl.BlockSpec((1, SDIM), lambda i: (0, 0)),
            pl.BlockSpec((SDIM, SDIM), lambda i: (0, 0)),
        ],
        out_specs=[
            pl.BlockSpec((BN, SDIM), lambda i: (i, 0)),
            pl.BlockSpec((BN, SDIM), lambda i: (i, 0)),
        ],
        out_shape=[
            jax.ShapeDtypeStruct((NP, SDIM), jnp.float32),
            jax.ShapeDtypeStruct((NP, SDIM), jnp.float32),
        ],
    )(s, agg, deg, uW1, ub1, uW2, ub2, lin_next)


def _final_body(s_ref, batch_ref, lng_ref, lnb_ref, post_ref,
                gsum_ref, gcnt_ref):
    step = pl.program_id(0)

    @pl.when(step == 0)
    def _():
        gsum_ref[...] = jnp.zeros_like(gsum_ref)
        gcnt_ref[...] = jnp.zeros_like(gcnt_ref)

    s = s_ref[...]
    mu = jnp.mean(s, axis=-1, keepdims=True)
    xc = s - mu
    var = jnp.mean(xc * xc, axis=-1, keepdims=True)
    sn = xc / jnp.sqrt(var + 1e-5) * lng_ref[...] + lnb_ref[...]
    p = jnp.dot(sn, post_ref[...], preferred_element_type=jnp.float32)
    bb = batch_ref[...]
    gio = lax.broadcasted_iota(jnp.int32, (BN, G), 1)
    oh = (gio == bb).astype(jnp.float32)
    part = lax.dot_general(oh, p, (((0,), (0,)), ((), ())),
                           preferred_element_type=jnp.float32)
    cnt = lax.dot_general(oh, jnp.ones((BN, SDIM), jnp.float32),
                          (((0,), (0,)), ((), ())),
                          preferred_element_type=jnp.float32)
    gsum_ref[...] += part
    gcnt_ref[...] += cnt


def _final_call(s, batch_p, lng, lnb, post_lin):
    grid = NP // BN
    return pl.pallas_call(
        _final_body,
        grid=(grid,),
        in_specs=[
            pl.BlockSpec((BN, SDIM), lambda i: (i, 0)),
            pl.BlockSpec((BN, 1), lambda i: (i, 0)),
            pl.BlockSpec((1, SDIM), lambda i: (0, 0)),
            pl.BlockSpec((1, SDIM), lambda i: (0, 0)),
            pl.BlockSpec((SDIM, SDIM), lambda i: (0, 0)),
        ],
        out_specs=[
            pl.BlockSpec((G, SDIM), lambda i: (0, 0)),
            pl.BlockSpec((G, SDIM), lambda i: (0, 0)),
        ],
        out_shape=[
            jax.ShapeDtypeStruct((G, SDIM), jnp.float32),
            jax.ShapeDtypeStruct((G, SDIM), jnp.float32),
        ],
    )(s, batch_p, lng, lnb, post_lin)


def _head_body(gsum_ref, gcnt_ref, d1W_ref, d1b_ref, d2W_ref, d2b_ref,
               a1W_ref, a1b_ref, a2W_ref, a2b_ref, out_ref):
    y = gsum_ref[...] / jnp.maximum(gcnt_ref[...], 1.0)
    y = _silu(jnp.dot(y, d1W_ref[...], preferred_element_type=jnp.float32)
              + d1b_ref[...])
    y = jnp.dot(y, d2W_ref[...], preferred_element_type=jnp.float32) + d2b_ref[...]
    a = _silu(jnp.dot(y, a1W_ref[...], preferred_element_type=jnp.float32)
              + a1b_ref[...])
    out_ref[...] = jnp.dot(a, a2W_ref[...],
                           preferred_element_type=jnp.float32) + a2b_ref[...]


def _head_call(gsum, gcnt, d1W, d1b, d2W, d2b, a1W, a1b, a2W_pad, a2b_pad):
    return pl.pallas_call(
        _head_body,
        out_shape=jax.ShapeDtypeStruct((G, 128), jnp.float32),
    )(gsum, gcnt, d1W, d1b, d2W, d2b, a1W, a1b, a2W_pad, a2b_pad)


@functools.cache
def _mesh():
    return plsc.VectorSubcoreMesh(core_axis_name="c", subcore_axis_name="s",
                                  num_cores=NSC, num_subcores=NSUB)


def _loc_indices(rowv, rowloc, off):
    @pl.loop(0, UNIT // 16)
    def _(j):
        r = rowv[pl.ds(j * 16, 16)]
        loc = r - off
        ok = (loc >= 0) & (loc < NHALF)
        rowloc[pl.ds(j * 16, 16)] = jnp.where(ok, loc, NHALF)


def _deg_kernel(row_hbm, out_hbm, rowv, rowloc, onesb, zb, acc):
    c = lax.axis_index("c")
    sid = lax.axis_index("s")
    off = c * NHALF

    zeros16 = jnp.zeros((16,), jnp.float32)

    @pl.loop(0, 16)
    def _(e):
        zb[e, :] = zeros16
        onesb[e, :] = zeros16 + 1.0

    @pl.loop(16, UNIT)
    def _(e):
        onesb[e, :] = zeros16 + 1.0

    @pl.loop(0, ROWS_PT // 16)
    def _(k):
        pltpu.sync_copy(zb, acc.at[pl.ds(sid * ROWS_PT + k * 16, 16)])

    plsc.subcore_barrier()

    @pl.loop(0, UNITS)
    def _(u):
        base = sid * EPT + u * UNIT
        pltpu.sync_copy(row_hbm.at[pl.ds(base, UNIT)], rowv)
        _loc_indices(rowv, rowloc, off)
        pltpu.sync_copy(onesb, acc.at[rowloc], add=True)

    plsc.subcore_barrier()
    pltpu.sync_copy(acc.at[pl.ds(sid * ROWS_PT, ROWS_PT)],
                    out_hbm.at[c, pl.ds(sid * ROWS_PT, ROWS_PT)])


def _deg_call(row_p):
    return pl.kernel(
        _deg_kernel,
        out_type=jax.ShapeDtypeStruct((NSC, ACC_ROWS, 16), jnp.float32),
        mesh=_mesh(),
        scratch_types=[
            pltpu.VMEM((UNIT,), jnp.int32),
            pltpu.VMEM((UNIT,), jnp.int32),
            pltpu.VMEM((UNIT, 16), jnp.float32),
            pltpu.VMEM((16, 16), jnp.float32),
            pltpu.VMEM_SHARED((ACC_ROWS, 16), jnp.float32),
        ],
        compiler_params=pltpu.CompilerParams(use_tc_tiling_on_sc=False),
    )(row_p)


def _msg_kernel(col_hbm, row_hbm, w_hbm, tab_hbm, out_hbm,
                colv, rowv, rowloc, gbuf, wbuf, zb, acc, sem):
    c = lax.axis_index("c")
    sid = lax.axis_index("s")
    off = c * NHALF

    zeros16 = jnp.zeros((16,), jnp.float32)

    @pl.loop(0, 16)
    def _(e):
        for j in range(SDIM // 16):
            zb[e, pl.ds(j * 16, 16)] = zeros16

    @pl.loop(0, ROWS_PT // 16)
    def _(k):
        pltpu.sync_copy(zb, acc.at[pl.ds(sid * ROWS_PT + k * 16, 16)])

    plsc.subcore_barrier()

    @pl.loop(0, UNITS)
    def _(u):
        base = sid * EPT + u * UNIT
        pltpu.sync_copy(col_hbm.at[pl.ds(base, UNIT)], colv)
        pltpu.sync_copy(row_hbm.at[pl.ds(base, UNIT)], rowv)
        pltpu.async_copy(tab_hbm.at[colv], gbuf, sem).wait()
        pltpu.sync_copy(w_hbm.at[pl.ds(base, UNIT)], wbuf)
        _loc_indices(rowv, rowloc, off)

        @pl.loop(0, UNIT)
        def _(e):
            for j in range(SDIM // 16):
                sl = pl.ds(j * 16, 16)
                gbuf[e, sl] = gbuf[e, sl] * wbuf[e, sl]

        pltpu.sync_copy(gbuf, acc.at[rowloc], add=True)

    plsc.subcore_barrier()
    pltpu.sync_copy(acc.at[pl.ds(sid * ROWS_PT, ROWS_PT)],
                    out_hbm.at[c, pl.ds(sid * ROWS_PT, ROWS_PT)])


def _msg_call(col_p, row_p, w, slin):
    return pl.kernel(
        _msg_kernel,
        out_type=jax.ShapeDtypeStruct((NSC, ACC_ROWS, SDIM), jnp.float32),
        mesh=_mesh(),
        scratch_types=[
            pltpu.VMEM((UNIT,), jnp.int32),
            pltpu.VMEM((UNIT,), jnp.int32),
            pltpu.VMEM((UNIT,), jnp.int32),
            pltpu.VMEM((UNIT, SDIM), jnp.float32),
            pltpu.VMEM((UNIT, SDIM), jnp.float32),
            pltpu.VMEM((16, SDIM), jnp.float32),
            pltpu.VMEM_SHARED((ACC_ROWS, SDIM), jnp.float32),
            pltpu.SemaphoreType.DMA,
        ],
        compiler_params=pltpu.CompilerParams(use_tc_tiling_on_sc=False),
    )(col_p, row_p, w, slin)


def kernel(x, pos, batch, edge_index, edge_weights, lig_flag, chains, params):
    del pos, lig_flag, chains
    row = edge_index[0]
    col = edge_index[1]
    row_p = jnp.pad(row, (0, E_PAD - E), constant_values=N)
    col_p = jnp.pad(col, (0, E_PAD - E), constant_values=0)
    d_p = jnp.pad(edge_weights, (0, E_PAD - E),
                  constant_values=1.0).reshape(E_PAD, 1)
    x_p = jnp.pad(x, (0, NP - N), constant_values=0).reshape(NP, 1)
    batch_p = jnp.pad(batch, (0, NP - N), constant_values=G).reshape(NP, 1)

    p = params
    emb_pad = jnp.pad(p['emb'], ((0, 128 - p['emb'].shape[0]), (0, 0)))
    layers = p['layers']

    def r1(v):
        return v.reshape(1, -1)

    s, slin = _embed_call(x_p, emb_pad, layers[0]['lin'])

    deg_out = _deg_call(row_p)
    deg = jnp.concatenate(
        [deg_out[0, :NHALF, 0], deg_out[1, :NHALF, 0],
         jnp.ones((NP - N,), jnp.float32)]).reshape(NP, 1)

    zlin = jnp.zeros((SDIM, SDIM), jnp.float32)
    for li, lp in enumerate(layers):
        w = _edge_w_call(d_p, lp['fW1'], r1(lp['fb1']), lp['fW2'], r1(lp['fb2']))
        agg_out = _msg_call(col_p, row_p, w, slin)
        agg = jnp.concatenate(
            [agg_out[0, :NHALF], agg_out[1, :NHALF],
             jnp.zeros((NP - N, SDIM), jnp.float32)], axis=0)
        lin_next = layers[li + 1]['lin'] if li + 1 < DEPTH else zlin
        s, slin = _update_call(s, agg, deg, lp['uW1'], r1(lp['ub1']),
                               lp['uW2'], r1(lp['ub2']), lin_next)

    gsum, gcnt = _final_call(s, batch_p, r1(p['ln_g']), r1(p['ln_b']),
                             p['post_lin'])

    a2W_pad = jnp.pad(p['a2W'], ((0, 0), (0, 128 - p['a2W'].shape[1])))
    a2b_pad = jnp.pad(p['a2b'], (0, 128 - p['a2b'].shape[0])).reshape(1, 128)
    out = _head_call(gsum, gcnt, p['d1W'], r1(p['d1b']), p['d2W'], r1(p['d2b']),
                     p['a1W'], r1(p['a1b']), a2W_pad, a2b_pad)
    return out[:, :1]
